# baseline (speedup 1.0000x reference)
import jax
import jax.numpy as jnp
from jax.experimental import pallas as pl
from jax.experimental.pallas import tpu as pltpu


def _fused_kernel(logits_ref, masks_ref, x_ref, W1_ref, b1_ref, W2_ref,
                  b2all_ref, Wg1_ref, bg1_ref, Wg2_ref, bg2_ref,
                  gamma_ref, beta_ref,
                  out_ref, guide_ref, gate_ref, h_ref, *, n_views, n_experts):
    ppv = n_experts // 2
    s = pl.program_id(0)
    p = jax.lax.rem(s, ppv)
    last = n_views * ppv - 1

    @pl.when(s == 0)
    def _init():
        out_ref[...] = jnp.zeros_like(out_ref)
        guide_ref[...] = jnp.zeros_like(guide_ref)

    @pl.when(p == 0)
    def _gates():
        logits = logits_ref[0]
        mask = masks_ref[0]
        probs = jax.nn.softmax(logits, axis=-1)
        gated = probs * mask
        gated = gated / (jnp.sum(gated, axis=-1, keepdims=True) + 1e-9)
        imp = jnp.mean(probs, axis=0, keepdims=True)
        load = jnp.mean(mask, axis=0, keepdims=True)
        guide_ref[...] += n_experts * jnp.sum(imp * load)

        @pl.when(s == 0)
        def _():
            gate_ref[:, 0:n_experts] = gated
            gate_ref[:, n_experts:2 * n_experts] = jnp.zeros_like(gated)

        @pl.when(s != 0)
        def _():
            gate_ref[:, n_experts:2 * n_experts] = gated

    gates = gate_ref[...]
    cols = jax.lax.broadcasted_iota(jnp.int32, (1, gates.shape[-1]), 1)
    g1 = jnp.sum(gates * (cols == 2 * s).astype(jnp.float32),
                 axis=-1, keepdims=True)
    g2 = jnp.sum(gates * (cols == 2 * s + 1).astype(jnp.float32),
                 axis=-1, keepdims=True)

    F = h_ref.shape[-1] // 2
    N = x_ref.shape[0]
    n_chunks = 2
    C = N // n_chunks
    W1a = W1_ref[0, 0]
    W1b = W1_ref[0, 1]
    W2p = W2_ref[0]

    for c in range(n_chunks):
        rows = pl.ds(c * C, C)
        xb = x_ref[rows, :].astype(jnp.bfloat16)
        g1b = g1[c * C:(c + 1) * C].astype(jnp.bfloat16)
        g2b = g2[c * C:(c + 1) * C].astype(jnp.bfloat16)
        h1 = jnp.dot(xb, W1a, preferred_element_type=jnp.float32)
        h1 = (h1 + b1_ref[0, 0]).astype(jnp.bfloat16)
        h_ref[rows, 0:F] = g1b * jax.nn.gelu(h1)
        h2 = jnp.dot(xb, W1b, preferred_element_type=jnp.float32)
        h2 = (h2 + b1_ref[0, 1]).astype(jnp.bfloat16)
        h_ref[rows, F:2 * F] = g2b * jax.nn.gelu(h2)
        out_ref[rows, :] += jnp.dot(h_ref[rows, :], W2p,
                                    preferred_element_type=jnp.float32)

    @pl.when(s == last)
    def _finish():
        for c in range(n_chunks):
            rows = pl.ds(c * C, C)
            x = x_ref[rows, :]
            bterm = jnp.dot(gate_ref[rows, :], b2all_ref[...],
                            preferred_element_type=jnp.float32)
            gh = jnp.dot(x.astype(jnp.bfloat16), Wg1_ref[...].astype(jnp.bfloat16),
                         preferred_element_type=jnp.float32)
            gh = jax.nn.gelu(gh + bg1_ref[0])
            gen = jnp.dot(gh.astype(jnp.bfloat16), Wg2_ref[...].astype(jnp.bfloat16),
                          preferred_element_type=jnp.float32)
            y = out_ref[rows, :] + bterm + gen + bg2_ref[0] + x
            mu = jnp.mean(y, axis=-1, keepdims=True)
            var = jnp.mean(jnp.square(y - mu), axis=-1, keepdims=True)
            out_ref[rows, :] = ((y - mu) * jax.lax.rsqrt(var + 1e-5)
                                * gamma_ref[0] + beta_ref[0])
        guide_ref[...] = guide_ref[...] / n_views


def kernel(x, total_logits, total_masks, W1, b1, W2, b2, Wg1, bg1, Wg2, bg2, gamma, beta):
    N, D = x.shape
    V, _, E = total_logits.shape
    F = W1.shape[-1]
    ppv = E // 2

    b1r = b1.reshape(V * ppv, 2, F)
    W2r = W2.reshape(V, E * F, D).astype(jnp.bfloat16)
    b2all = b2.reshape(V * E, D)
    W1c = W1.astype(jnp.bfloat16)

    grid = (V * ppv,)
    out, guide = pl.pallas_call(
        lambda *refs: _fused_kernel(*refs, n_views=V, n_experts=E),
        grid=grid,
        in_specs=[
            pl.BlockSpec((1, N, E), lambda s: (s // ppv, 0, 0)),
            pl.BlockSpec((1, N, E), lambda s: (s // ppv, 0, 0)),
            pl.BlockSpec((N, D), lambda s: (0, 0)),
            pl.BlockSpec((1, 2, D, F), lambda s: (s // ppv, s % ppv, 0, 0)),
            pl.BlockSpec((1, 2, F), lambda s: (s, 0, 0)),
            pl.BlockSpec((1, 2 * F, D), lambda s: (s // ppv, s % ppv, 0)),
            pl.BlockSpec((V * E, D), lambda s: (0, 0)),
            pl.BlockSpec((D, F), lambda s: (0, 0)),
            pl.BlockSpec((1, F), lambda s: (0, 0)),
            pl.BlockSpec((F, D), lambda s: (0, 0)),
            pl.BlockSpec((1, D), lambda s: (0, 0)),
            pl.BlockSpec((1, D), lambda s: (0, 0)),
            pl.BlockSpec((1, D), lambda s: (0, 0)),
        ],
        out_specs=[
            pl.BlockSpec((N, D), lambda s: (0, 0)),
            pl.BlockSpec((1, 1), lambda s: (0, 0)),
        ],
        out_shape=[
            jax.ShapeDtypeStruct((N, D), jnp.float32),
            jax.ShapeDtypeStruct((1, 1), jnp.float32),
        ],
        scratch_shapes=[
            pltpu.VMEM((N, V * E), jnp.float32),
            pltpu.VMEM((N, 2 * F), jnp.bfloat16),
        ],
        compiler_params=pltpu.CompilerParams(
            dimension_semantics=("arbitrary",),
        ),
    )(total_logits, total_masks, x, W1c, b1r, W2r, b2all,
      Wg1, bg1.reshape(1, F), Wg2, bg2.reshape(1, D),
      gamma.reshape(1, D), beta.reshape(1, D))
    return out, guide[0, 0]


# speedup vs baseline: 1.3010x; 1.3010x over previous
import jax
import jax.numpy as jnp
from jax.experimental import pallas as pl
from jax.experimental.pallas import tpu as pltpu


def _fused_kernel(logits_ref, masks_ref, x_ref, W1_ref, b1_ref, W2_ref,
                  b2all_ref, Wg1_ref, bg1_ref, Wg2_ref, bg2_ref,
                  gamma_ref, beta_ref,
                  out_ref, guide_ref, gate_ref, h_ref, *, n_views, n_experts):
    ppv = n_experts // 2
    s = pl.program_id(0)
    p = jax.lax.rem(s, ppv)
    last = n_views * ppv - 1

    @pl.when(s == 0)
    def _init():
        out_ref[...] = jnp.zeros_like(out_ref)
        guide_ref[...] = jnp.zeros_like(guide_ref)

    @pl.when(p == 0)
    def _gates():
        logits = logits_ref[0]
        mask = masks_ref[0]
        probs = jax.nn.softmax(logits, axis=-1)
        gated = probs * mask
        gated = gated / (jnp.sum(gated, axis=-1, keepdims=True) + 1e-9)
        imp = jnp.mean(probs, axis=0, keepdims=True)
        load = jnp.mean(mask, axis=0, keepdims=True)
        guide_ref[...] += n_experts * jnp.sum(imp * load)

        @pl.when(s == 0)
        def _():
            gate_ref[:, 0:n_experts] = gated
            gate_ref[:, n_experts:2 * n_experts] = jnp.zeros_like(gated)

        @pl.when(s != 0)
        def _():
            gate_ref[:, n_experts:2 * n_experts] = gated

    gates = gate_ref[...]
    cols = jax.lax.broadcasted_iota(jnp.int32, (1, gates.shape[-1]), 1)
    g1 = jnp.sum(gates * (cols == 2 * s).astype(jnp.float32),
                 axis=-1, keepdims=True)
    g2 = jnp.sum(gates * (cols == 2 * s + 1).astype(jnp.float32),
                 axis=-1, keepdims=True)

    F = h_ref.shape[-1] // 2
    N = x_ref.shape[0]
    n_chunks = 2
    C = N // n_chunks
    W1a = W1_ref[0, 0].astype(jnp.bfloat16)
    W1b = W1_ref[0, 1].astype(jnp.bfloat16)
    W2p = W2_ref[0].astype(jnp.bfloat16)

    for c in range(n_chunks):
        rows = pl.ds(c * C, C)
        xb = x_ref[rows, :].astype(jnp.bfloat16)
        g1b = g1[c * C:(c + 1) * C].astype(jnp.bfloat16)
        g2b = g2[c * C:(c + 1) * C].astype(jnp.bfloat16)
        h1 = jnp.dot(xb, W1a, preferred_element_type=jnp.float32)
        h1 = (h1 + b1_ref[0, 0]).astype(jnp.bfloat16)
        h_ref[rows, 0:F] = g1b * jax.nn.gelu(h1)
        h2 = jnp.dot(xb, W1b, preferred_element_type=jnp.float32)
        h2 = (h2 + b1_ref[0, 1]).astype(jnp.bfloat16)
        h_ref[rows, F:2 * F] = g2b * jax.nn.gelu(h2)
        out_ref[rows, :] += jnp.dot(h_ref[rows, :], W2p,
                                    preferred_element_type=jnp.float32)

    @pl.when(s == last)
    def _finish():
        for c in range(n_chunks):
            rows = pl.ds(c * C, C)
            x = x_ref[rows, :]
            bterm = jnp.dot(gate_ref[rows, :], b2all_ref[...],
                            preferred_element_type=jnp.float32)
            gh = jnp.dot(x.astype(jnp.bfloat16), Wg1_ref[...].astype(jnp.bfloat16),
                         preferred_element_type=jnp.float32)
            gh = jax.nn.gelu(gh + bg1_ref[0])
            gen = jnp.dot(gh.astype(jnp.bfloat16), Wg2_ref[...].astype(jnp.bfloat16),
                          preferred_element_type=jnp.float32)
            y = out_ref[rows, :] + bterm + gen + bg2_ref[0] + x
            mu = jnp.mean(y, axis=-1, keepdims=True)
            var = jnp.mean(jnp.square(y - mu), axis=-1, keepdims=True)
            out_ref[rows, :] = ((y - mu) * jax.lax.rsqrt(var + 1e-5)
                                * gamma_ref[0] + beta_ref[0])
        guide_ref[...] = guide_ref[...] / n_views


def kernel(x, total_logits, total_masks, W1, b1, W2, b2, Wg1, bg1, Wg2, bg2, gamma, beta):
    N, D = x.shape
    V, _, E = total_logits.shape
    F = W1.shape[-1]
    ppv = E // 2

    b1r = b1.reshape(V * ppv, 2, F)
    W2r = W2.reshape(V, E * F, D)
    b2all = b2.reshape(V * E, D)

    grid = (V * ppv,)
    out, guide = pl.pallas_call(
        lambda *refs: _fused_kernel(*refs, n_views=V, n_experts=E),
        grid=grid,
        in_specs=[
            pl.BlockSpec((1, N, E), lambda s: (s // ppv, 0, 0)),
            pl.BlockSpec((1, N, E), lambda s: (s // ppv, 0, 0)),
            pl.BlockSpec((N, D), lambda s: (0, 0)),
            pl.BlockSpec((1, 2, D, F), lambda s: (s // ppv, s % ppv, 0, 0)),
            pl.BlockSpec((1, 2, F), lambda s: (s, 0, 0)),
            pl.BlockSpec((1, 2 * F, D), lambda s: (s // ppv, s % ppv, 0)),
            pl.BlockSpec((V * E, D), lambda s: (0, 0)),
            pl.BlockSpec((D, F), lambda s: (0, 0)),
            pl.BlockSpec((1, F), lambda s: (0, 0)),
            pl.BlockSpec((F, D), lambda s: (0, 0)),
            pl.BlockSpec((1, D), lambda s: (0, 0)),
            pl.BlockSpec((1, D), lambda s: (0, 0)),
            pl.BlockSpec((1, D), lambda s: (0, 0)),
        ],
        out_specs=[
            pl.BlockSpec((N, D), lambda s: (0, 0)),
            pl.BlockSpec((1, 1), lambda s: (0, 0)),
        ],
        out_shape=[
            jax.ShapeDtypeStruct((N, D), jnp.float32),
            jax.ShapeDtypeStruct((1, 1), jnp.float32),
        ],
        scratch_shapes=[
            pltpu.VMEM((N, V * E), jnp.float32),
            pltpu.VMEM((N, 2 * F), jnp.bfloat16),
        ],
        compiler_params=pltpu.CompilerParams(
            dimension_semantics=("arbitrary",),
        ),
    )(total_logits, total_masks, x, W1, b1r, W2r, b2all,
      Wg1, bg1.reshape(1, F), Wg2, bg2.reshape(1, D),
      gamma.reshape(1, D), beta.reshape(1, D))
    return out, guide[0, 0]
